# baseline (speedup 1.0000x reference)
import functools

import jax
import jax.numpy as jnp
from jax import lax
from jax.experimental import pallas as pl
from jax.experimental.pallas import tpu as pltpu
from jax.experimental.pallas import tpu_sc as plsc

N = 10000
E = 320000
D = 128
G = 64

NC = 2
NS = 16
NW = NC * NS
CH = 128
QT = 79
PCHUNK = NW * QT
PAD_E = PCHUNK * CH - E
NPAD = 8
ZR = 200
NZB = N // ZR
OUT_SUB = 40
CNT_W = 16

_mesh = plsc.VectorSubcoreMesh(core_axis_name="c", subcore_axis_name="s")
_SC_PARAMS = pltpu.CompilerParams(use_tc_tiling_on_sc=False)


def _over_blocks(s, fn):
    def body(b, _):
        cid = s + b * NS

        @pl.when(cid < NZB)
        def _():
            fn(pl.multiple_of(cid * ZR, ZR))

        return 0

    lax.fori_loop(0, (NZB + NS - 1) // NS, body, 0)


def _sc_body(with_cnt, x_hbm, src_hbm, dst_hbm, *rest):
    if with_cnt:
        (out_hbm, cnt_hbm, agg_sh, cnt_sh, src_v, dst_v, rows_v, ones_v,
         zcnt_v, sem) = rest
    else:
        (out_hbm, agg_sh, src_v, dst_v, rows_v, sem) = rest
    c = lax.axis_index("c")
    s = lax.axis_index("s")
    wid = s * NC + c

    zf = jnp.zeros((16,), jnp.float32)

    def fill_zrow(i, _):
        for j in range(D // 16):
            rows_v[i, pl.ds(j * 16, 16)] = zf
        return 0

    lax.fori_loop(0, CH, fill_zrow, 0)

    if with_cnt:
        one = jnp.ones((16,), jnp.float32)

        def fill_small(i, _):
            ones_v[i, :] = one

            @pl.when(i < OUT_SUB)
            def _():
                zcnt_v[i, :] = zf

            return 0

        lax.fori_loop(0, CH, fill_small, 0)

    def do_zero(r0):
        pltpu.sync_copy(rows_v.at[pl.ds(0, CH)], agg_sh.at[pl.ds(r0, CH)])
        pltpu.sync_copy(rows_v.at[pl.ds(0, ZR - CH)],
                        agg_sh.at[pl.ds(r0 + CH, ZR - CH)])
        if with_cnt:
            def sub(j, _):
                rr = pl.multiple_of(r0 + j * OUT_SUB, 8)
                pltpu.sync_copy(zcnt_v, cnt_sh.at[pl.ds(rr, OUT_SUB)])
                return 0

            lax.fori_loop(0, ZR // OUT_SUB, sub, 0)

    _over_blocks(s, do_zero)

    @pl.when(s == 0)
    def _():
        pltpu.sync_copy(rows_v.at[pl.ds(0, NPAD)], agg_sh.at[pl.ds(N, NPAD)])
        if with_cnt:
            pltpu.sync_copy(zcnt_v.at[pl.ds(0, NPAD)],
                            cnt_sh.at[pl.ds(N, NPAD)])

    plsc.subcore_barrier()

    def chunk_body(k, _):
        ci = wid + k * NW
        pltpu.sync_copy(src_hbm.at[ci], src_v)
        pltpu.sync_copy(dst_hbm.at[ci], dst_v)
        pltpu.async_copy(x_hbm.at[src_v], rows_v, sem).wait()
        pltpu.sync_copy(rows_v, agg_sh.at[dst_v], add=True)
        if with_cnt:
            pltpu.sync_copy(ones_v, cnt_sh.at[dst_v], add=True)
        return 0

    lax.fori_loop(0, QT, chunk_body, 0)

    plsc.subcore_barrier()

    def do_out(r0):
        def sub(j, _):
            rr = pl.multiple_of(r0 + j * OUT_SUB, 8)
            pltpu.sync_copy(agg_sh.at[pl.ds(rr, OUT_SUB)],
                            out_hbm.at[c, pl.ds(rr, OUT_SUB)])
            if with_cnt:
                pltpu.sync_copy(cnt_sh.at[pl.ds(rr, OUT_SUB)],
                                cnt_hbm.at[c, pl.ds(rr, OUT_SUB)])
            return 0

        lax.fori_loop(0, ZR // OUT_SUB, sub, 0)

    _over_blocks(s, do_out)


_sc_agg_cnt = pl.kernel(
    functools.partial(_sc_body, True),
    out_type=(jax.ShapeDtypeStruct((NC, N, D), jnp.float32),
              jax.ShapeDtypeStruct((NC, N, CNT_W), jnp.float32)),
    mesh=_mesh,
    compiler_params=_SC_PARAMS,
    scratch_types=[
        pltpu.VMEM_SHARED((N + NPAD, D), jnp.float32),
        pltpu.VMEM_SHARED((N + NPAD, CNT_W), jnp.float32),
        pltpu.VMEM((CH,), jnp.int32),
        pltpu.VMEM((CH,), jnp.int32),
        pltpu.VMEM((CH, D), jnp.float32),
        pltpu.VMEM((CH, CNT_W), jnp.float32),
        pltpu.VMEM((OUT_SUB, CNT_W), jnp.float32),
        pltpu.SemaphoreType.DMA,
    ],
)

_sc_agg = pl.kernel(
    functools.partial(_sc_body, False),
    out_type=jax.ShapeDtypeStruct((NC, N, D), jnp.float32),
    mesh=_mesh,
    compiler_params=_SC_PARAMS,
    scratch_types=[
        pltpu.VMEM_SHARED((N + NPAD, D), jnp.float32),
        pltpu.VMEM((CH,), jnp.int32),
        pltpu.VMEM((CH,), jnp.int32),
        pltpu.VMEM((CH, D), jnp.float32),
        pltpu.SemaphoreType.DMA,
    ],
)


TB = 1000
TGRID = N // TB


def _tc_layer_body(relu, p_ref, cnt_ref, x_ref, wl_ref, wr_ref, b_ref, o_ref):
    cnt = cnt_ref[0][:, 0:1] + cnt_ref[1][:, 0:1]
    agg = (p_ref[0] + p_ref[1]) * (1.0 / jnp.maximum(cnt, 1.0))
    h = (jnp.dot(agg, wl_ref[...], preferred_element_type=jnp.float32)
         + jnp.dot(x_ref[...], wr_ref[...], preferred_element_type=jnp.float32)
         + b_ref[...])
    o_ref[...] = jnp.maximum(h, 0.0) if relu else h


def _tc_layer(p, cnt, x, wlT, wrT, bl, relu):
    return pl.pallas_call(
        functools.partial(_tc_layer_body, relu),
        grid=(TGRID,),
        in_specs=[
            pl.BlockSpec((NC, TB, D), lambda i: (0, i, 0)),
            pl.BlockSpec((NC, TB, CNT_W), lambda i: (0, i, 0)),
            pl.BlockSpec((TB, D), lambda i: (i, 0)),
            pl.BlockSpec((D, D), lambda i: (0, 0)),
            pl.BlockSpec((D, D), lambda i: (0, 0)),
            pl.BlockSpec((1, D), lambda i: (0, 0)),
        ],
        out_specs=pl.BlockSpec((TB, D), lambda i: (i, 0)),
        out_shape=jax.ShapeDtypeStruct((N, D), jnp.float32),
    )(p, cnt, x, wlT, wrT, bl)


def _tc_pool_body(p_ref, cnt_ref, x_ref, seg_ref, wl_ref, wr_ref, b_ref,
                  o_ref, acc, cac):
    i = pl.program_id(0)
    cnt = cnt_ref[0][:, 0:1] + cnt_ref[1][:, 0:1]
    agg = (p_ref[0] + p_ref[1]) * (1.0 / jnp.maximum(cnt, 1.0))
    h = (jnp.dot(agg, wl_ref[...], preferred_element_type=jnp.float32)
         + jnp.dot(x_ref[...], wr_ref[...], preferred_element_type=jnp.float32)
         + b_ref[...])
    oh = (seg_ref[...] == lax.broadcasted_iota(jnp.int32, (TB, G), 1)
          ).astype(jnp.float32)
    dn = (((0,), (0,)), ((), ()))
    part = lax.dot_general(oh, h, dn, preferred_element_type=jnp.float32)
    pcnt = lax.dot_general(oh, jnp.ones((TB, D), jnp.float32), dn,
                           preferred_element_type=jnp.float32)

    @pl.when(i == 0)
    def _():
        acc[...] = jnp.zeros((G, D), jnp.float32)
        cac[...] = jnp.zeros((G, D), jnp.float32)

    acc[...] += part
    cac[...] += pcnt

    @pl.when(i == TGRID - 1)
    def _():
        o_ref[...] = acc[...] / jnp.maximum(cac[...], 1.0)


def _tc_pool(p, cnt, x, seg, wlT, wrT, bl):
    return pl.pallas_call(
        _tc_pool_body,
        grid=(TGRID,),
        in_specs=[
            pl.BlockSpec((NC, TB, D), lambda i: (0, i, 0)),
            pl.BlockSpec((NC, TB, CNT_W), lambda i: (0, i, 0)),
            pl.BlockSpec((TB, D), lambda i: (i, 0)),
            pl.BlockSpec((TB, 1), lambda i: (i, 0)),
            pl.BlockSpec((D, D), lambda i: (0, 0)),
            pl.BlockSpec((D, D), lambda i: (0, 0)),
            pl.BlockSpec((1, D), lambda i: (0, 0)),
        ],
        out_specs=pl.BlockSpec((G, D), lambda i: (0, 0)),
        out_shape=jax.ShapeDtypeStruct((G, D), jnp.float32),
        scratch_shapes=[
            pltpu.VMEM((G, D), jnp.float32),
            pltpu.VMEM((G, D), jnp.float32),
        ],
    )(p, cnt, x, seg, wlT, wrT, bl)


def kernel(x, edge_index, batch, edge_attr,
           Wl0, bl0, Wr0, Wl1, bl1, Wr1, Wl2, bl2, Wr2):
    x = x.astype(jnp.float32)
    ei = edge_index.astype(jnp.int32)
    src = jnp.concatenate(
        [ei[0], jnp.zeros((PAD_E,), jnp.int32)]).reshape(PCHUNK, CH)
    dst = jnp.concatenate(
        [ei[1], jnp.full((PAD_E,), N, jnp.int32)]).reshape(PCHUNK, CH)
    seg = batch.astype(jnp.int32).reshape(N, 1)

    p, cnt = _sc_agg_cnt(x, src, dst)
    x1 = _tc_layer(p, cnt, x, Wl0.T, Wr0.T, bl0.reshape(1, D), relu=True)
    p = _sc_agg(x1, src, dst)
    x2 = _tc_layer(p, cnt, x1, Wl1.T, Wr1.T, bl1.reshape(1, D), relu=True)
    p = _sc_agg(x2, src, dst)
    return _tc_pool(p, cnt, x2, seg, Wl2.T, Wr2.T, bl2.reshape(1, D))


# speedup vs baseline: 1.4255x; 1.4255x over previous
import functools

import jax
import jax.numpy as jnp
from jax import lax
from jax.experimental import pallas as pl
from jax.experimental.pallas import tpu as pltpu
from jax.experimental.pallas import tpu_sc as plsc

N = 10000
E = 320000
D = 128
G = 64

NC = 2
NS = 16
NW = NC * NS
CH = 128
QT = 79
PCHUNK = NW * QT
PAD_E = PCHUNK * CH - E
NPAD = 128
ZR = 200
NZB = N // ZR
OUT_SUB = 40
CNT_W = 16

_mesh = plsc.VectorSubcoreMesh(core_axis_name="c", subcore_axis_name="s")
_SC_PARAMS = pltpu.CompilerParams(use_tc_tiling_on_sc=False)


def _over_blocks(s, fn):
    def body(b, _):
        cid = s + b * NS

        @pl.when(cid < NZB)
        def _():
            fn(pl.multiple_of(cid * ZR, ZR))

        return 0

    lax.fori_loop(0, (NZB + NS - 1) // NS, body, 0)


def _sc_body(with_cnt, x_hbm, src_hbm, dst_hbm, *rest):
    if with_cnt:
        (out_hbm, cnt_hbm, agg_sh, cnt_sh, src_v, dst_v, rows_v, ones_v,
         zcnt_v, sem) = rest
    else:
        (out_hbm, agg_sh, src_v, dst_v, rows_v, sem) = rest
    c = lax.axis_index("c")
    s = lax.axis_index("s")
    wid = s * NC + c

    zf = jnp.zeros((16,), jnp.float32)

    def fill_zrow(i, _):
        for j in range(D // 16):
            rows_v[i, pl.ds(j * 16, 16)] = zf
        return 0

    lax.fori_loop(0, CH, fill_zrow, 0)

    if with_cnt:
        one = jnp.ones((16,), jnp.float32)

        def fill_small(i, _):
            ones_v[i, :] = one

            @pl.when(i < OUT_SUB)
            def _():
                zcnt_v[i, :] = zf

            return 0

        lax.fori_loop(0, CH, fill_small, 0)

    def do_zero(r0):
        pltpu.sync_copy(rows_v.at[pl.ds(0, CH)], agg_sh.at[pl.ds(r0, CH)])
        pltpu.sync_copy(rows_v.at[pl.ds(0, ZR - CH)],
                        agg_sh.at[pl.ds(r0 + CH, ZR - CH)])
        if with_cnt:
            def sub(j, _):
                rr = pl.multiple_of(r0 + j * OUT_SUB, 8)
                pltpu.sync_copy(zcnt_v, cnt_sh.at[pl.ds(rr, OUT_SUB)])
                return 0

            lax.fori_loop(0, ZR // OUT_SUB, sub, 0)

    _over_blocks(s, do_zero)

    plsc.subcore_barrier()

    def chunk_body(k, _):
        ci = wid + k * NW
        pltpu.sync_copy(src_hbm.at[ci], src_v)
        pltpu.sync_copy(dst_hbm.at[ci], dst_v)
        pltpu.async_copy(x_hbm.at[src_v], rows_v, sem).wait()
        pltpu.sync_copy(rows_v, agg_sh.at[dst_v], add=True)
        if with_cnt:
            pltpu.sync_copy(ones_v, cnt_sh.at[dst_v], add=True)
        return 0

    lax.fori_loop(0, QT, chunk_body, 0)

    plsc.subcore_barrier()

    def do_out(r0):
        def sub(j, _):
            rr = pl.multiple_of(r0 + j * OUT_SUB, 8)
            pltpu.sync_copy(agg_sh.at[pl.ds(rr, OUT_SUB)],
                            out_hbm.at[c, pl.ds(rr, OUT_SUB)])
            if with_cnt:
                pltpu.sync_copy(cnt_sh.at[pl.ds(rr, OUT_SUB)],
                                cnt_hbm.at[c, pl.ds(rr, OUT_SUB)])
            return 0

        lax.fori_loop(0, ZR // OUT_SUB, sub, 0)

    _over_blocks(s, do_out)


_sc_agg_cnt = pl.kernel(
    functools.partial(_sc_body, True),
    out_type=(jax.ShapeDtypeStruct((NC, N, D), jnp.float32),
              jax.ShapeDtypeStruct((NC, N, CNT_W), jnp.float32)),
    mesh=_mesh,
    compiler_params=_SC_PARAMS,
    scratch_types=[
        pltpu.VMEM_SHARED((N + NPAD, D), jnp.float32),
        pltpu.VMEM_SHARED((N + NPAD, CNT_W), jnp.float32),
        pltpu.VMEM((CH,), jnp.int32),
        pltpu.VMEM((CH,), jnp.int32),
        pltpu.VMEM((CH, D), jnp.float32),
        pltpu.VMEM((CH, CNT_W), jnp.float32),
        pltpu.VMEM((OUT_SUB, CNT_W), jnp.float32),
        pltpu.SemaphoreType.DMA,
    ],
)

_sc_agg = pl.kernel(
    functools.partial(_sc_body, False),
    out_type=jax.ShapeDtypeStruct((NC, N, D), jnp.float32),
    mesh=_mesh,
    compiler_params=_SC_PARAMS,
    scratch_types=[
        pltpu.VMEM_SHARED((N + NPAD, D), jnp.float32),
        pltpu.VMEM((CH,), jnp.int32),
        pltpu.VMEM((CH,), jnp.int32),
        pltpu.VMEM((CH, D), jnp.float32),
        pltpu.SemaphoreType.DMA,
    ],
)


TB = 1000
TGRID = N // TB


def _tc_layer_body(relu, p_ref, cnt_ref, x_ref, wl_ref, wr_ref, b_ref, o_ref):
    cnt = cnt_ref[0][:, 0:1] + cnt_ref[1][:, 0:1]
    agg = (p_ref[0] + p_ref[1]) * (1.0 / jnp.maximum(cnt, 1.0))
    h = (jnp.dot(agg, wl_ref[...], preferred_element_type=jnp.float32)
         + jnp.dot(x_ref[...], wr_ref[...], preferred_element_type=jnp.float32)
         + b_ref[...])
    o_ref[...] = jnp.maximum(h, 0.0) if relu else h


def _tc_layer(p, cnt, x, wlT, wrT, bl, relu):
    return pl.pallas_call(
        functools.partial(_tc_layer_body, relu),
        grid=(TGRID,),
        in_specs=[
            pl.BlockSpec((NC, TB, D), lambda i: (0, i, 0)),
            pl.BlockSpec((NC, TB, CNT_W), lambda i: (0, i, 0)),
            pl.BlockSpec((TB, D), lambda i: (i, 0)),
            pl.BlockSpec((D, D), lambda i: (0, 0)),
            pl.BlockSpec((D, D), lambda i: (0, 0)),
            pl.BlockSpec((1, D), lambda i: (0, 0)),
        ],
        out_specs=pl.BlockSpec((TB, D), lambda i: (i, 0)),
        out_shape=jax.ShapeDtypeStruct((N, D), jnp.float32),
    )(p, cnt, x, wlT, wrT, bl)


def _tc_pool_body(p_ref, cnt_ref, x_ref, seg_ref, wl_ref, wr_ref, b_ref,
                  o_ref, acc, cac):
    i = pl.program_id(0)
    cnt = cnt_ref[0][:, 0:1] + cnt_ref[1][:, 0:1]
    agg = (p_ref[0] + p_ref[1]) * (1.0 / jnp.maximum(cnt, 1.0))
    h = (jnp.dot(agg, wl_ref[...], preferred_element_type=jnp.float32)
         + jnp.dot(x_ref[...], wr_ref[...], preferred_element_type=jnp.float32)
         + b_ref[...])
    oh = (seg_ref[...] == lax.broadcasted_iota(jnp.int32, (TB, G), 1)
          ).astype(jnp.float32)
    dn = (((0,), (0,)), ((), ()))
    part = lax.dot_general(oh, h, dn, preferred_element_type=jnp.float32)
    pcnt = lax.dot_general(oh, jnp.ones((TB, D), jnp.float32), dn,
                           preferred_element_type=jnp.float32)

    @pl.when(i == 0)
    def _():
        acc[...] = jnp.zeros((G, D), jnp.float32)
        cac[...] = jnp.zeros((G, D), jnp.float32)

    acc[...] += part
    cac[...] += pcnt

    @pl.when(i == TGRID - 1)
    def _():
        o_ref[...] = acc[...] / jnp.maximum(cac[...], 1.0)


def _tc_pool(p, cnt, x, seg, wlT, wrT, bl):
    return pl.pallas_call(
        _tc_pool_body,
        grid=(TGRID,),
        in_specs=[
            pl.BlockSpec((NC, TB, D), lambda i: (0, i, 0)),
            pl.BlockSpec((NC, TB, CNT_W), lambda i: (0, i, 0)),
            pl.BlockSpec((TB, D), lambda i: (i, 0)),
            pl.BlockSpec((TB, 1), lambda i: (i, 0)),
            pl.BlockSpec((D, D), lambda i: (0, 0)),
            pl.BlockSpec((D, D), lambda i: (0, 0)),
            pl.BlockSpec((1, D), lambda i: (0, 0)),
        ],
        out_specs=pl.BlockSpec((G, D), lambda i: (0, 0)),
        out_shape=jax.ShapeDtypeStruct((G, D), jnp.float32),
        scratch_shapes=[
            pltpu.VMEM((G, D), jnp.float32),
            pltpu.VMEM((G, D), jnp.float32),
        ],
    )(p, cnt, x, seg, wlT, wrT, bl)


def kernel(x, edge_index, batch, edge_attr,
           Wl0, bl0, Wr0, Wl1, bl1, Wr1, Wl2, bl2, Wr2):
    x = x.astype(jnp.float32)
    ei = edge_index.astype(jnp.int32)
    stripe = jnp.arange(PAD_E, dtype=jnp.int32)
    src = jnp.concatenate(
        [ei[0], stripe % jnp.int32(N)]).reshape(PCHUNK, CH)
    dst = jnp.concatenate(
        [ei[1], jnp.int32(N) + stripe % jnp.int32(NPAD)]).reshape(PCHUNK, CH)
    seg = batch.astype(jnp.int32).reshape(N, 1)

    p, cnt = _sc_agg_cnt(x, src, dst)
    x1 = _tc_layer(p, cnt, x, Wl0.T, Wr0.T, bl0.reshape(1, D), relu=True)
    p = _sc_agg(x1, src, dst)
    x2 = _tc_layer(p, cnt, x1, Wl1.T, Wr1.T, bl1.reshape(1, D), relu=True)
    p = _sc_agg(x2, src, dst)
    return _tc_pool(p, cnt, x2, seg, Wl2.T, Wr2.T, bl2.reshape(1, D))


# speedup vs baseline: 1.6101x; 1.1295x over previous
import functools

import jax
import jax.numpy as jnp
from jax import lax
from jax.experimental import pallas as pl
from jax.experimental.pallas import tpu as pltpu
from jax.experimental.pallas import tpu_sc as plsc

N = 10000
E = 320000
D = 128
G = 64

NC = 2
NS = 16
NW = NC * NS
CH = 128
QT = 79
PCHUNK = NW * QT
PAD_E = PCHUNK * CH - E
NPAD = 128
ZR = 200
NZB = N // ZR
OUT_SUB = 40
CNT_W = 16

_mesh = plsc.VectorSubcoreMesh(core_axis_name="c", subcore_axis_name="s")
_SC_PARAMS = pltpu.CompilerParams(use_tc_tiling_on_sc=False)


def _over_blocks(s, fn):
    def body(b, _):
        cid = s + b * NS

        @pl.when(cid < NZB)
        def _():
            fn(pl.multiple_of(cid * ZR, ZR))

        return 0

    lax.fori_loop(0, (NZB + NS - 1) // NS, body, 0)


def _sc_body(with_cnt, x_hbm, idx_hbm, *rest):
    if with_cnt:
        (out_hbm, cnt_hbm, agg_sh, cnt_sh, idx_v, rows_v, ones_v,
         zcnt_v, sem) = rest
    else:
        (out_hbm, agg_sh, idx_v, rows_v, sem) = rest
    c = lax.axis_index("c")
    s = lax.axis_index("s")
    wid = s * NC + c

    zf = jnp.zeros((16,), jnp.float32)

    def fill_zrow(i, _):
        for j in range(D // 16):
            rows_v[i, pl.ds(j * 16, 16)] = zf
        return 0

    lax.fori_loop(0, CH, fill_zrow, 0)

    if with_cnt:
        one = jnp.ones((16,), jnp.float32)

        def fill_small(i, _):
            ones_v[i, :] = one

            @pl.when(i < OUT_SUB)
            def _():
                zcnt_v[i, :] = zf

            return 0

        lax.fori_loop(0, CH, fill_small, 0)

    def do_zero(r0):
        pltpu.sync_copy(rows_v.at[pl.ds(0, CH)], agg_sh.at[pl.ds(r0, CH)])
        pltpu.sync_copy(rows_v.at[pl.ds(0, ZR - CH)],
                        agg_sh.at[pl.ds(r0 + CH, ZR - CH)])
        if with_cnt:
            def sub(j, _):
                rr = pl.multiple_of(r0 + j * OUT_SUB, 8)
                pltpu.sync_copy(zcnt_v, cnt_sh.at[pl.ds(rr, OUT_SUB)])
                return 0

            lax.fori_loop(0, ZR // OUT_SUB, sub, 0)

    _over_blocks(s, do_zero)

    plsc.subcore_barrier()

    def chunk_body(k, _):
        ci = wid + k * NW
        pltpu.sync_copy(idx_hbm.at[ci], idx_v)
        pltpu.async_copy(x_hbm.at[idx_v.at[0]], rows_v, sem).wait()
        pltpu.sync_copy(rows_v, agg_sh.at[idx_v.at[1]], add=True)
        if with_cnt:
            pltpu.sync_copy(ones_v, cnt_sh.at[idx_v.at[1]], add=True)
        return 0

    lax.fori_loop(0, QT, chunk_body, 0)

    plsc.subcore_barrier()

    def do_out(r0):
        def sub(j, _):
            rr = pl.multiple_of(r0 + j * OUT_SUB, 8)
            pltpu.sync_copy(agg_sh.at[pl.ds(rr, OUT_SUB)],
                            out_hbm.at[c, pl.ds(rr, OUT_SUB)])
            if with_cnt:
                pltpu.sync_copy(cnt_sh.at[pl.ds(rr, OUT_SUB)],
                                cnt_hbm.at[c, pl.ds(rr, OUT_SUB)])
            return 0

        lax.fori_loop(0, ZR // OUT_SUB, sub, 0)

    _over_blocks(s, do_out)


_sc_agg_cnt = pl.kernel(
    functools.partial(_sc_body, True),
    out_type=(jax.ShapeDtypeStruct((NC, N, D), jnp.float32),
              jax.ShapeDtypeStruct((NC, N, CNT_W), jnp.float32)),
    mesh=_mesh,
    compiler_params=_SC_PARAMS,
    scratch_types=[
        pltpu.VMEM_SHARED((N + NPAD, D), jnp.float32),
        pltpu.VMEM_SHARED((N + NPAD, CNT_W), jnp.float32),
        pltpu.VMEM((2, CH), jnp.int32),
        pltpu.VMEM((CH, D), jnp.float32),
        pltpu.VMEM((CH, CNT_W), jnp.float32),
        pltpu.VMEM((OUT_SUB, CNT_W), jnp.float32),
        pltpu.SemaphoreType.DMA,
    ],
)

_sc_agg = pl.kernel(
    functools.partial(_sc_body, False),
    out_type=jax.ShapeDtypeStruct((NC, N, D), jnp.float32),
    mesh=_mesh,
    compiler_params=_SC_PARAMS,
    scratch_types=[
        pltpu.VMEM_SHARED((N + NPAD, D), jnp.float32),
        pltpu.VMEM((2, CH), jnp.int32),
        pltpu.VMEM((CH, D), jnp.float32),
        pltpu.SemaphoreType.DMA,
    ],
)


TB = 1000
TGRID = N // TB


def _tc_layer_body(relu, p_ref, cnt_ref, x_ref, wl_ref, wr_ref, b_ref, o_ref):
    cnt = cnt_ref[0][:, 0:1] + cnt_ref[1][:, 0:1]
    agg = (p_ref[0] + p_ref[1]) * (1.0 / jnp.maximum(cnt, 1.0))
    h = (jnp.dot(agg, wl_ref[...], preferred_element_type=jnp.float32)
         + jnp.dot(x_ref[...], wr_ref[...], preferred_element_type=jnp.float32)
         + b_ref[...])
    o_ref[...] = jnp.maximum(h, 0.0) if relu else h


def _tc_layer(p, cnt, x, wlT, wrT, bl, relu):
    return pl.pallas_call(
        functools.partial(_tc_layer_body, relu),
        grid=(TGRID,),
        in_specs=[
            pl.BlockSpec((NC, TB, D), lambda i: (0, i, 0)),
            pl.BlockSpec((NC, TB, CNT_W), lambda i: (0, i, 0)),
            pl.BlockSpec((TB, D), lambda i: (i, 0)),
            pl.BlockSpec((D, D), lambda i: (0, 0)),
            pl.BlockSpec((D, D), lambda i: (0, 0)),
            pl.BlockSpec((1, D), lambda i: (0, 0)),
        ],
        out_specs=pl.BlockSpec((TB, D), lambda i: (i, 0)),
        out_shape=jax.ShapeDtypeStruct((N, D), jnp.float32),
    )(p, cnt, x, wlT, wrT, bl)


def _tc_pool_body(p_ref, cnt_ref, x_ref, seg_ref, wl_ref, wr_ref, b_ref,
                  o_ref, acc, cac):
    i = pl.program_id(0)
    cnt = cnt_ref[0][:, 0:1] + cnt_ref[1][:, 0:1]
    agg = (p_ref[0] + p_ref[1]) * (1.0 / jnp.maximum(cnt, 1.0))
    h = (jnp.dot(agg, wl_ref[...], preferred_element_type=jnp.float32)
         + jnp.dot(x_ref[...], wr_ref[...], preferred_element_type=jnp.float32)
         + b_ref[...])
    oh = (seg_ref[...] == lax.broadcasted_iota(jnp.int32, (TB, G), 1)
          ).astype(jnp.float32)
    dn = (((0,), (0,)), ((), ()))
    part = lax.dot_general(oh, h, dn, preferred_element_type=jnp.float32)
    pcnt = lax.dot_general(oh, jnp.ones((TB, D), jnp.float32), dn,
                           preferred_element_type=jnp.float32)

    @pl.when(i == 0)
    def _():
        acc[...] = jnp.zeros((G, D), jnp.float32)
        cac[...] = jnp.zeros((G, D), jnp.float32)

    acc[...] += part
    cac[...] += pcnt

    @pl.when(i == TGRID - 1)
    def _():
        o_ref[...] = acc[...] / jnp.maximum(cac[...], 1.0)


def _tc_pool(p, cnt, x, seg, wlT, wrT, bl):
    return pl.pallas_call(
        _tc_pool_body,
        grid=(TGRID,),
        in_specs=[
            pl.BlockSpec((NC, TB, D), lambda i: (0, i, 0)),
            pl.BlockSpec((NC, TB, CNT_W), lambda i: (0, i, 0)),
            pl.BlockSpec((TB, D), lambda i: (i, 0)),
            pl.BlockSpec((TB, 1), lambda i: (i, 0)),
            pl.BlockSpec((D, D), lambda i: (0, 0)),
            pl.BlockSpec((D, D), lambda i: (0, 0)),
            pl.BlockSpec((1, D), lambda i: (0, 0)),
        ],
        out_specs=pl.BlockSpec((G, D), lambda i: (0, 0)),
        out_shape=jax.ShapeDtypeStruct((G, D), jnp.float32),
        scratch_shapes=[
            pltpu.VMEM((G, D), jnp.float32),
            pltpu.VMEM((G, D), jnp.float32),
        ],
    )(p, cnt, x, seg, wlT, wrT, bl)


def kernel(x, edge_index, batch, edge_attr,
           Wl0, bl0, Wr0, Wl1, bl1, Wr1, Wl2, bl2, Wr2):
    x = x.astype(jnp.float32)
    ei = edge_index.astype(jnp.int32)
    stripe = jnp.arange(PAD_E, dtype=jnp.int32)
    src = jnp.concatenate(
        [ei[0], stripe % jnp.int32(N)]).reshape(PCHUNK, 1, CH)
    dst = jnp.concatenate(
        [ei[1], jnp.int32(N) + stripe % jnp.int32(NPAD)]).reshape(PCHUNK, 1, CH)
    idx = jnp.concatenate([src, dst], axis=1)
    seg = batch.astype(jnp.int32).reshape(N, 1)

    p, cnt = _sc_agg_cnt(x, idx)
    x1 = _tc_layer(p, cnt, x, Wl0.T, Wr0.T, bl0.reshape(1, D), relu=True)
    p = _sc_agg(x1, idx)
    x2 = _tc_layer(p, cnt, x1, Wl1.T, Wr1.T, bl1.reshape(1, D), relu=True)
    p = _sc_agg(x2, idx)
    return _tc_pool(p, cnt, x2, seg, Wl2.T, Wr2.T, bl2.reshape(1, D))


# speedup vs baseline: 1.7912x; 1.1125x over previous
import functools

import jax
import jax.numpy as jnp
from jax import lax
from jax.experimental import pallas as pl
from jax.experimental.pallas import tpu as pltpu
from jax.experimental.pallas import tpu_sc as plsc

N = 10000
E = 320000
D = 128
G = 64

NC = 2
NS = 16
NW = NC * NS
CH = 128
QT = 80
PCHUNK = NW * QT
IB = 4
QB = QT // IB
PAD_E = PCHUNK * CH - E
NPAD = 128
ZR = 200
NZB = N // ZR
OUT_SUB = 40
CNT_W = 16

_mesh = plsc.VectorSubcoreMesh(core_axis_name="c", subcore_axis_name="s")
_SC_PARAMS = pltpu.CompilerParams(use_tc_tiling_on_sc=False)


def _over_blocks(s, fn):
    def body(b, _):
        cid = s + b * NS

        @pl.when(cid < NZB)
        def _():
            fn(pl.multiple_of(cid * ZR, ZR))

        return 0

    lax.fori_loop(0, (NZB + NS - 1) // NS, body, 0)


def _sc_body(with_cnt, x_hbm, idx_hbm, *rest):
    if with_cnt:
        (out_hbm, cnt_hbm, agg_sh, cnt_sh, idx_v, rows_v, ones_v,
         zcnt_v, sem) = rest
    else:
        (out_hbm, agg_sh, idx_v, rows_v, sem) = rest
    c = lax.axis_index("c")
    s = lax.axis_index("s")
    wid = s * NC + c

    zf = jnp.zeros((16,), jnp.float32)

    def fill_zrow(i, _):
        for j in range(D // 16):
            rows_v[i, pl.ds(j * 16, 16)] = zf
        return 0

    lax.fori_loop(0, CH, fill_zrow, 0)

    if with_cnt:
        one = jnp.ones((16,), jnp.float32)

        def fill_small(i, _):
            ones_v[i, :] = one

            @pl.when(i < OUT_SUB)
            def _():
                zcnt_v[i, :] = zf

            return 0

        lax.fori_loop(0, CH, fill_small, 0)

    def do_zero(r0):
        pltpu.sync_copy(rows_v.at[pl.ds(0, CH)], agg_sh.at[pl.ds(r0, CH)])
        pltpu.sync_copy(rows_v.at[pl.ds(0, ZR - CH)],
                        agg_sh.at[pl.ds(r0 + CH, ZR - CH)])
        if with_cnt:
            def sub(j, _):
                rr = pl.multiple_of(r0 + j * OUT_SUB, 8)
                pltpu.sync_copy(zcnt_v, cnt_sh.at[pl.ds(rr, OUT_SUB)])
                return 0

            lax.fori_loop(0, ZR // OUT_SUB, sub, 0)

    _over_blocks(s, do_zero)

    plsc.subcore_barrier()

    def chunk_block(u, _):
        bid = wid + u * NW
        pltpu.sync_copy(idx_hbm.at[bid], idx_v)
        for j in range(IB):
            pltpu.async_copy(x_hbm.at[idx_v.at[j, 0]], rows_v, sem).wait()
            pltpu.sync_copy(rows_v, agg_sh.at[idx_v.at[j, 1]], add=True)
            if with_cnt:
                pltpu.sync_copy(ones_v, cnt_sh.at[idx_v.at[j, 1]], add=True)
        return 0

    lax.fori_loop(0, QB, chunk_block, 0)

    plsc.subcore_barrier()

    def do_out(r0):
        def sub(j, _):
            rr = pl.multiple_of(r0 + j * OUT_SUB, 8)
            pltpu.sync_copy(agg_sh.at[pl.ds(rr, OUT_SUB)],
                            out_hbm.at[c, pl.ds(rr, OUT_SUB)])
            if with_cnt:
                pltpu.sync_copy(cnt_sh.at[pl.ds(rr, OUT_SUB)],
                                cnt_hbm.at[c, pl.ds(rr, OUT_SUB)])
            return 0

        lax.fori_loop(0, ZR // OUT_SUB, sub, 0)

    _over_blocks(s, do_out)


_sc_agg_cnt = pl.kernel(
    functools.partial(_sc_body, True),
    out_type=(jax.ShapeDtypeStruct((NC, N, D), jnp.float32),
              jax.ShapeDtypeStruct((NC, N, CNT_W), jnp.float32)),
    mesh=_mesh,
    compiler_params=_SC_PARAMS,
    scratch_types=[
        pltpu.VMEM_SHARED((N + NPAD, D), jnp.float32),
        pltpu.VMEM_SHARED((N + NPAD, CNT_W), jnp.float32),
        pltpu.VMEM((IB, 2, CH), jnp.int32),
        pltpu.VMEM((CH, D), jnp.float32),
        pltpu.VMEM((CH, CNT_W), jnp.float32),
        pltpu.VMEM((OUT_SUB, CNT_W), jnp.float32),
        pltpu.SemaphoreType.DMA,
    ],
)

_sc_agg = pl.kernel(
    functools.partial(_sc_body, False),
    out_type=jax.ShapeDtypeStruct((NC, N, D), jnp.float32),
    mesh=_mesh,
    compiler_params=_SC_PARAMS,
    scratch_types=[
        pltpu.VMEM_SHARED((N + NPAD, D), jnp.float32),
        pltpu.VMEM((IB, 2, CH), jnp.int32),
        pltpu.VMEM((CH, D), jnp.float32),
        pltpu.SemaphoreType.DMA,
    ],
)


TB = 1000
TGRID = N // TB


def _tc_layer_body(relu, p_ref, cnt_ref, x_ref, wl_ref, wr_ref, b_ref, o_ref):
    cnt = cnt_ref[0][:, 0:1] + cnt_ref[1][:, 0:1]
    agg = (p_ref[0] + p_ref[1]) * (1.0 / jnp.maximum(cnt, 1.0))
    h = (jnp.dot(agg, wl_ref[...], preferred_element_type=jnp.float32)
         + jnp.dot(x_ref[...], wr_ref[...], preferred_element_type=jnp.float32)
         + b_ref[...])
    o_ref[...] = jnp.maximum(h, 0.0) if relu else h


def _tc_layer(p, cnt, x, wlT, wrT, bl, relu):
    return pl.pallas_call(
        functools.partial(_tc_layer_body, relu),
        grid=(TGRID,),
        in_specs=[
            pl.BlockSpec((NC, TB, D), lambda i: (0, i, 0)),
            pl.BlockSpec((NC, TB, CNT_W), lambda i: (0, i, 0)),
            pl.BlockSpec((TB, D), lambda i: (i, 0)),
            pl.BlockSpec((D, D), lambda i: (0, 0)),
            pl.BlockSpec((D, D), lambda i: (0, 0)),
            pl.BlockSpec((1, D), lambda i: (0, 0)),
        ],
        out_specs=pl.BlockSpec((TB, D), lambda i: (i, 0)),
        out_shape=jax.ShapeDtypeStruct((N, D), jnp.float32),
    )(p, cnt, x, wlT, wrT, bl)


def _tc_pool_body(p_ref, cnt_ref, x_ref, seg_ref, wl_ref, wr_ref, b_ref,
                  o_ref, acc, cac):
    i = pl.program_id(0)
    cnt = cnt_ref[0][:, 0:1] + cnt_ref[1][:, 0:1]
    agg = (p_ref[0] + p_ref[1]) * (1.0 / jnp.maximum(cnt, 1.0))
    h = (jnp.dot(agg, wl_ref[...], preferred_element_type=jnp.float32)
         + jnp.dot(x_ref[...], wr_ref[...], preferred_element_type=jnp.float32)
         + b_ref[...])
    oh = (seg_ref[...] == lax.broadcasted_iota(jnp.int32, (TB, G), 1)
          ).astype(jnp.float32)
    dn = (((0,), (0,)), ((), ()))
    part = lax.dot_general(oh, h, dn, preferred_element_type=jnp.float32)
    pcnt = lax.dot_general(oh, jnp.ones((TB, D), jnp.float32), dn,
                           preferred_element_type=jnp.float32)

    @pl.when(i == 0)
    def _():
        acc[...] = jnp.zeros((G, D), jnp.float32)
        cac[...] = jnp.zeros((G, D), jnp.float32)

    acc[...] += part
    cac[...] += pcnt

    @pl.when(i == TGRID - 1)
    def _():
        o_ref[...] = acc[...] / jnp.maximum(cac[...], 1.0)


def _tc_pool(p, cnt, x, seg, wlT, wrT, bl):
    return pl.pallas_call(
        _tc_pool_body,
        grid=(TGRID,),
        in_specs=[
            pl.BlockSpec((NC, TB, D), lambda i: (0, i, 0)),
            pl.BlockSpec((NC, TB, CNT_W), lambda i: (0, i, 0)),
            pl.BlockSpec((TB, D), lambda i: (i, 0)),
            pl.BlockSpec((TB, 1), lambda i: (i, 0)),
            pl.BlockSpec((D, D), lambda i: (0, 0)),
            pl.BlockSpec((D, D), lambda i: (0, 0)),
            pl.BlockSpec((1, D), lambda i: (0, 0)),
        ],
        out_specs=pl.BlockSpec((G, D), lambda i: (0, 0)),
        out_shape=jax.ShapeDtypeStruct((G, D), jnp.float32),
        scratch_shapes=[
            pltpu.VMEM((G, D), jnp.float32),
            pltpu.VMEM((G, D), jnp.float32),
        ],
    )(p, cnt, x, seg, wlT, wrT, bl)


def kernel(x, edge_index, batch, edge_attr,
           Wl0, bl0, Wr0, Wl1, bl1, Wr1, Wl2, bl2, Wr2):
    x = x.astype(jnp.float32)
    ei = edge_index.astype(jnp.int32)
    stripe = jnp.arange(PAD_E, dtype=jnp.int32)
    src = jnp.concatenate(
        [ei[0], stripe % jnp.int32(N)]).reshape(PCHUNK, 1, CH)
    dst = jnp.concatenate(
        [ei[1], jnp.int32(N) + stripe % jnp.int32(NPAD)]).reshape(PCHUNK, 1, CH)
    idx = jnp.concatenate(
        [src, dst], axis=1).reshape(PCHUNK // IB, IB, 2, CH)
    seg = batch.astype(jnp.int32).reshape(N, 1)

    p, cnt = _sc_agg_cnt(x, idx)
    x1 = _tc_layer(p, cnt, x, Wl0.T, Wr0.T, bl0.reshape(1, D), relu=True)
    p = _sc_agg(x1, idx)
    x2 = _tc_layer(p, cnt, x1, Wl1.T, Wr1.T, bl1.reshape(1, D), relu=True)
    p = _sc_agg(x2, idx)
    return _tc_pool(p, cnt, x2, seg, Wl2.T, Wr2.T, bl2.reshape(1, D))


# speedup vs baseline: 2.1934x; 1.2245x over previous
import functools

import jax
import jax.numpy as jnp
from jax import lax
from jax.experimental import pallas as pl
from jax.experimental.pallas import tpu as pltpu
from jax.experimental.pallas import tpu_sc as plsc

N = 10000
E = 320000
D = 128
G = 64

NC = 2
NS = 16
NW = NC * NS
CH = 128
QT = 80
PCHUNK = NW * QT
IB = 8
QB = QT // IB
PAD_E = PCHUNK * CH - E
NPAD = 128
ZR = 200
NZB = N // ZR
OUT_SUB = 40
CNT_W = 16

_mesh = plsc.VectorSubcoreMesh(core_axis_name="c", subcore_axis_name="s")
_SC_PARAMS = pltpu.CompilerParams(use_tc_tiling_on_sc=False)


def _over_blocks(s, fn):
    def body(b, _):
        cid = s + b * NS

        @pl.when(cid < NZB)
        def _():
            fn(pl.multiple_of(cid * ZR, ZR))

        return 0

    lax.fori_loop(0, (NZB + NS - 1) // NS, body, 0)


def _sc_body(with_cnt, x_hbm, idx_hbm, *rest):
    if with_cnt:
        (out_hbm, cnt_hbm, agg_sh, cnt_sh, idx_v, rows_v, ones_v,
         zcnt_v, sem) = rest
    else:
        (out_hbm, agg_sh, idx_v, rows_v, sem) = rest
    c = lax.axis_index("c")
    s = lax.axis_index("s")
    wid = s * NC + c

    zf = jnp.zeros((16,), jnp.float32)

    def fill_zrow(i, _):
        for j in range(D // 16):
            rows_v[i, pl.ds(j * 16, 16)] = zf
        return 0

    lax.fori_loop(0, CH, fill_zrow, 0)

    if with_cnt:
        one = jnp.ones((16,), jnp.float32)

        def fill_small(i, _):
            ones_v[i, :] = one

            @pl.when(i < OUT_SUB)
            def _():
                zcnt_v[i, :] = zf

            return 0

        lax.fori_loop(0, CH, fill_small, 0)

    def do_zero(r0):
        pltpu.sync_copy(rows_v.at[pl.ds(0, CH)], agg_sh.at[pl.ds(r0, CH)])
        pltpu.sync_copy(rows_v.at[pl.ds(0, ZR - CH)],
                        agg_sh.at[pl.ds(r0 + CH, ZR - CH)])
        if with_cnt:
            def sub(j, _):
                rr = pl.multiple_of(r0 + j * OUT_SUB, 8)
                pltpu.sync_copy(zcnt_v, cnt_sh.at[pl.ds(rr, OUT_SUB)])
                return 0

            lax.fori_loop(0, ZR // OUT_SUB, sub, 0)

    _over_blocks(s, do_zero)

    plsc.subcore_barrier()

    def chunk_block(u, _):
        bid = wid + u * NW
        pltpu.sync_copy(idx_hbm.at[bid], idx_v)
        for j in range(IB):
            pltpu.async_copy(x_hbm.at[idx_v.at[j, 0]], rows_v, sem).wait()
            pltpu.sync_copy(rows_v, agg_sh.at[idx_v.at[j, 1]], add=True)
            if with_cnt:
                pltpu.sync_copy(ones_v, cnt_sh.at[idx_v.at[j, 1]], add=True)
        return 0

    lax.fori_loop(0, QB, chunk_block, 0)

    plsc.subcore_barrier()

    def do_out(r0):
        def sub(j, _):
            rr = pl.multiple_of(r0 + j * OUT_SUB, 8)
            pltpu.sync_copy(agg_sh.at[pl.ds(rr, OUT_SUB)],
                            out_hbm.at[c, pl.ds(rr, OUT_SUB)])
            if with_cnt:
                pltpu.sync_copy(cnt_sh.at[pl.ds(rr, OUT_SUB)],
                                cnt_hbm.at[c, pl.ds(rr, OUT_SUB)])
            return 0

        lax.fori_loop(0, ZR // OUT_SUB, sub, 0)

    _over_blocks(s, do_out)


_sc_agg_cnt = pl.kernel(
    functools.partial(_sc_body, True),
    out_type=(jax.ShapeDtypeStruct((NC, N, D), jnp.float32),
              jax.ShapeDtypeStruct((NC, N, CNT_W), jnp.float32)),
    mesh=_mesh,
    compiler_params=_SC_PARAMS,
    scratch_types=[
        pltpu.VMEM_SHARED((N + NPAD, D), jnp.float32),
        pltpu.VMEM_SHARED((N + NPAD, CNT_W), jnp.float32),
        pltpu.VMEM((IB, 2, CH), jnp.int32),
        pltpu.VMEM((CH, D), jnp.float32),
        pltpu.VMEM((CH, CNT_W), jnp.float32),
        pltpu.VMEM((OUT_SUB, CNT_W), jnp.float32),
        pltpu.SemaphoreType.DMA,
    ],
)

def _sc_body_dbuf(x_hbm, idx_hbm, out_hbm, agg_sh, idx_v, rows0, rows1,
                  sem0, sem1):
    c = lax.axis_index("c")
    s = lax.axis_index("s")
    wid = s * NC + c

    zf = jnp.zeros((16,), jnp.float32)

    def fill_zrow(i, _):
        for j in range(D // 16):
            rows0[i, pl.ds(j * 16, 16)] = zf
        return 0

    lax.fori_loop(0, CH, fill_zrow, 0)

    def do_zero(r0):
        pltpu.sync_copy(rows0.at[pl.ds(0, CH)], agg_sh.at[pl.ds(r0, CH)])
        pltpu.sync_copy(rows0.at[pl.ds(0, ZR - CH)],
                        agg_sh.at[pl.ds(r0 + CH, ZR - CH)])

    _over_blocks(s, do_zero)
    plsc.subcore_barrier()

    bufs = ((rows0, sem0), (rows1, sem1))

    def chunk_block(u, _):
        bid = wid + u * NW
        pltpu.sync_copy(idx_hbm.at[bid], idx_v)
        pltpu.async_copy(x_hbm.at[idx_v.at[0, 0]], rows0, sem0)
        for j in range(IB):
            rv, sm = bufs[j % 2]
            if j + 1 < IB:
                nrv, nsm = bufs[(j + 1) % 2]
                pltpu.async_copy(x_hbm.at[idx_v.at[j + 1, 0]], nrv, nsm)
            pltpu.make_async_copy(x_hbm.at[idx_v.at[j, 0]], rv, sm).wait()
            pltpu.sync_copy(rv, agg_sh.at[idx_v.at[j, 1]], add=True)
        return 0

    lax.fori_loop(0, QB, chunk_block, 0)

    plsc.subcore_barrier()

    def do_out(r0):
        def sub(j, _):
            rr = pl.multiple_of(r0 + j * OUT_SUB, 8)
            pltpu.sync_copy(agg_sh.at[pl.ds(rr, OUT_SUB)],
                            out_hbm.at[c, pl.ds(rr, OUT_SUB)])
            return 0

        lax.fori_loop(0, ZR // OUT_SUB, sub, 0)

    _over_blocks(s, do_out)


_sc_agg = pl.kernel(
    _sc_body_dbuf,
    out_type=jax.ShapeDtypeStruct((NC, N, D), jnp.float32),
    mesh=_mesh,
    compiler_params=_SC_PARAMS,
    scratch_types=[
        pltpu.VMEM_SHARED((N + NPAD, D), jnp.float32),
        pltpu.VMEM((IB, 2, CH), jnp.int32),
        pltpu.VMEM((CH, D), jnp.float32),
        pltpu.VMEM((CH, D), jnp.float32),
        pltpu.SemaphoreType.DMA,
        pltpu.SemaphoreType.DMA,
    ],
)


TB = 1000
TGRID = N // TB


def _tc_layer_body(relu, p_ref, cnt_ref, x_ref, wl_ref, wr_ref, b_ref, o_ref):
    cnt = cnt_ref[0][:, 0:1] + cnt_ref[1][:, 0:1]
    agg = (p_ref[0] + p_ref[1]) * (1.0 / jnp.maximum(cnt, 1.0))
    h = (jnp.dot(agg, wl_ref[...], preferred_element_type=jnp.float32)
         + jnp.dot(x_ref[...], wr_ref[...], preferred_element_type=jnp.float32)
         + b_ref[...])
    o_ref[...] = jnp.maximum(h, 0.0) if relu else h


def _tc_layer(p, cnt, x, wlT, wrT, bl, relu):
    return pl.pallas_call(
        functools.partial(_tc_layer_body, relu),
        grid=(TGRID,),
        in_specs=[
            pl.BlockSpec((NC, TB, D), lambda i: (0, i, 0)),
            pl.BlockSpec((NC, TB, CNT_W), lambda i: (0, i, 0)),
            pl.BlockSpec((TB, D), lambda i: (i, 0)),
            pl.BlockSpec((D, D), lambda i: (0, 0)),
            pl.BlockSpec((D, D), lambda i: (0, 0)),
            pl.BlockSpec((1, D), lambda i: (0, 0)),
        ],
        out_specs=pl.BlockSpec((TB, D), lambda i: (i, 0)),
        out_shape=jax.ShapeDtypeStruct((N, D), jnp.float32),
    )(p, cnt, x, wlT, wrT, bl)


def _tc_pool_body(p_ref, cnt_ref, x_ref, seg_ref, wl_ref, wr_ref, b_ref,
                  o_ref, acc, cac):
    i = pl.program_id(0)
    cnt = cnt_ref[0][:, 0:1] + cnt_ref[1][:, 0:1]
    agg = (p_ref[0] + p_ref[1]) * (1.0 / jnp.maximum(cnt, 1.0))
    h = (jnp.dot(agg, wl_ref[...], preferred_element_type=jnp.float32)
         + jnp.dot(x_ref[...], wr_ref[...], preferred_element_type=jnp.float32)
         + b_ref[...])
    oh = (seg_ref[...] == lax.broadcasted_iota(jnp.int32, (TB, G), 1)
          ).astype(jnp.float32)
    dn = (((0,), (0,)), ((), ()))
    part = lax.dot_general(oh, h, dn, preferred_element_type=jnp.float32)
    pcnt = lax.dot_general(oh, jnp.ones((TB, D), jnp.float32), dn,
                           preferred_element_type=jnp.float32)

    @pl.when(i == 0)
    def _():
        acc[...] = jnp.zeros((G, D), jnp.float32)
        cac[...] = jnp.zeros((G, D), jnp.float32)

    acc[...] += part
    cac[...] += pcnt

    @pl.when(i == TGRID - 1)
    def _():
        o_ref[...] = acc[...] / jnp.maximum(cac[...], 1.0)


def _tc_pool(p, cnt, x, seg, wlT, wrT, bl):
    return pl.pallas_call(
        _tc_pool_body,
        grid=(TGRID,),
        in_specs=[
            pl.BlockSpec((NC, TB, D), lambda i: (0, i, 0)),
            pl.BlockSpec((NC, TB, CNT_W), lambda i: (0, i, 0)),
            pl.BlockSpec((TB, D), lambda i: (i, 0)),
            pl.BlockSpec((TB, 1), lambda i: (i, 0)),
            pl.BlockSpec((D, D), lambda i: (0, 0)),
            pl.BlockSpec((D, D), lambda i: (0, 0)),
            pl.BlockSpec((1, D), lambda i: (0, 0)),
        ],
        out_specs=pl.BlockSpec((G, D), lambda i: (0, 0)),
        out_shape=jax.ShapeDtypeStruct((G, D), jnp.float32),
        scratch_shapes=[
            pltpu.VMEM((G, D), jnp.float32),
            pltpu.VMEM((G, D), jnp.float32),
        ],
    )(p, cnt, x, seg, wlT, wrT, bl)


def kernel(x, edge_index, batch, edge_attr,
           Wl0, bl0, Wr0, Wl1, bl1, Wr1, Wl2, bl2, Wr2):
    x = x.astype(jnp.float32)
    ei = edge_index.astype(jnp.int32)
    stripe = jnp.arange(PAD_E, dtype=jnp.int32)
    src = jnp.concatenate(
        [ei[0], stripe % jnp.int32(N)]).reshape(PCHUNK, 1, CH)
    dst = jnp.concatenate(
        [ei[1], jnp.int32(N) + stripe % jnp.int32(NPAD)]).reshape(PCHUNK, 1, CH)
    idx = jnp.concatenate(
        [src, dst], axis=1).reshape(PCHUNK // IB, IB, 2, CH)
    seg = batch.astype(jnp.int32).reshape(N, 1)

    p, cnt = _sc_agg_cnt(x, idx)
    x1 = _tc_layer(p, cnt, x, Wl0.T, Wr0.T, bl0.reshape(1, D), relu=True)
    p = _sc_agg(x1, idx)
    x2 = _tc_layer(p, cnt, x1, Wl1.T, Wr1.T, bl1.reshape(1, D), relu=True)
    p = _sc_agg(x2, idx)
    return _tc_pool(p, cnt, x2, seg, Wl2.T, Wr2.T, bl2.reshape(1, D))


# speedup vs baseline: 2.4139x; 1.1005x over previous
import functools

import jax
import jax.numpy as jnp
from jax import lax
from jax.experimental import pallas as pl
from jax.experimental.pallas import tpu as pltpu
from jax.experimental.pallas import tpu_sc as plsc

N = 10000
E = 320000
D = 128
G = 64

NC = 2
NS = 16
NW = NC * NS
CH = 128
QT = 80
PCHUNK = NW * QT
IB = 8
QB = QT // IB
PAD_E = PCHUNK * CH - E
NPAD = 128
ZR = 200
NZB = N // ZR
OUT_SUB = 40
CNT_W = 16

_mesh = plsc.VectorSubcoreMesh(core_axis_name="c", subcore_axis_name="s")
_SC_PARAMS = pltpu.CompilerParams(use_tc_tiling_on_sc=False)


def _over_blocks(s, fn):
    def body(b, _):
        cid = s + b * NS

        @pl.when(cid < NZB)
        def _():
            fn(pl.multiple_of(cid * ZR, ZR))

        return 0

    lax.fori_loop(0, (NZB + NS - 1) // NS, body, 0)


def _sc_cnt_body(idx_hbm, cnt_hbm, cnt_sh, idx_v, ones_v, zcnt_v):
    c = lax.axis_index("c")
    s = lax.axis_index("s")
    wid = s * NC + c

    zf = jnp.zeros((16,), jnp.float32)
    one = jnp.ones((16,), jnp.float32)

    def fill(i, _):
        ones_v[i, :] = one

        @pl.when(i < OUT_SUB)
        def _():
            zcnt_v[i, :] = zf

        return 0

    lax.fori_loop(0, CH, fill, 0)

    def do_zero(r0):
        def sub(j, _):
            rr = pl.multiple_of(r0 + j * OUT_SUB, 8)
            pltpu.sync_copy(zcnt_v, cnt_sh.at[pl.ds(rr, OUT_SUB)])
            return 0

        lax.fori_loop(0, ZR // OUT_SUB, sub, 0)

    _over_blocks(s, do_zero)
    plsc.subcore_barrier()

    def chunk_block(u, _):
        bid = wid + u * NW
        pltpu.sync_copy(idx_hbm.at[bid], idx_v)
        for j in range(IB):
            pltpu.sync_copy(ones_v, cnt_sh.at[idx_v.at[j, 1]], add=True)
        return 0

    lax.fori_loop(0, QB, chunk_block, 0)

    plsc.subcore_barrier()

    def do_out(r0):
        def sub(j, _):
            rr = pl.multiple_of(r0 + j * OUT_SUB, 8)
            pltpu.sync_copy(cnt_sh.at[pl.ds(rr, OUT_SUB)],
                            cnt_hbm.at[c, pl.ds(rr, OUT_SUB)])
            return 0

        lax.fori_loop(0, ZR // OUT_SUB, sub, 0)

    _over_blocks(s, do_out)


_sc_cnt = pl.kernel(
    _sc_cnt_body,
    out_type=jax.ShapeDtypeStruct((NC, N, CNT_W), jnp.float32),
    mesh=_mesh,
    compiler_params=_SC_PARAMS,
    scratch_types=[
        pltpu.VMEM_SHARED((N + NPAD, CNT_W), jnp.float32),
        pltpu.VMEM((IB, 2, CH), jnp.int32),
        pltpu.VMEM((CH, CNT_W), jnp.float32),
        pltpu.VMEM((OUT_SUB, CNT_W), jnp.float32),
    ],
)

def _sc_body_dbuf(x_hbm, idx_hbm, out_hbm, agg_sh, idx_v, rows0, rows1,
                  sem0, sem1):
    c = lax.axis_index("c")
    s = lax.axis_index("s")
    wid = s * NC + c

    zf = jnp.zeros((16,), jnp.float32)

    def fill_zrow(i, _):
        for j in range(D // 16):
            rows0[i, pl.ds(j * 16, 16)] = zf
        return 0

    lax.fori_loop(0, CH, fill_zrow, 0)

    def do_zero(r0):
        pltpu.sync_copy(rows0.at[pl.ds(0, CH)], agg_sh.at[pl.ds(r0, CH)])
        pltpu.sync_copy(rows0.at[pl.ds(0, ZR - CH)],
                        agg_sh.at[pl.ds(r0 + CH, ZR - CH)])

    _over_blocks(s, do_zero)
    plsc.subcore_barrier()

    bufs = ((rows0, sem0), (rows1, sem1))

    def chunk_block(u, _):
        bid = wid + u * NW
        pltpu.sync_copy(idx_hbm.at[bid], idx_v)
        pltpu.async_copy(x_hbm.at[idx_v.at[0, 0]], rows0, sem0)
        for j in range(IB):
            rv, sm = bufs[j % 2]
            if j + 1 < IB:
                nrv, nsm = bufs[(j + 1) % 2]
                pltpu.async_copy(x_hbm.at[idx_v.at[j + 1, 0]], nrv, nsm)
            pltpu.make_async_copy(x_hbm.at[idx_v.at[j, 0]], rv, sm).wait()
            pltpu.sync_copy(rv, agg_sh.at[idx_v.at[j, 1]], add=True)
        return 0

    lax.fori_loop(0, QB, chunk_block, 0)

    plsc.subcore_barrier()

    def do_out(r0):
        def sub(j, _):
            rr = pl.multiple_of(r0 + j * OUT_SUB, 8)
            pltpu.sync_copy(agg_sh.at[pl.ds(rr, OUT_SUB)],
                            out_hbm.at[c, pl.ds(rr, OUT_SUB)])
            return 0

        lax.fori_loop(0, ZR // OUT_SUB, sub, 0)

    _over_blocks(s, do_out)


_sc_agg = pl.kernel(
    _sc_body_dbuf,
    out_type=jax.ShapeDtypeStruct((NC, N, D), jnp.float32),
    mesh=_mesh,
    compiler_params=_SC_PARAMS,
    scratch_types=[
        pltpu.VMEM_SHARED((N + NPAD, D), jnp.float32),
        pltpu.VMEM((IB, 2, CH), jnp.int32),
        pltpu.VMEM((CH, D), jnp.float32),
        pltpu.VMEM((CH, D), jnp.float32),
        pltpu.SemaphoreType.DMA,
        pltpu.SemaphoreType.DMA,
    ],
)


TB = 1000
TGRID = N // TB


def _tc_layer_body(relu, p_ref, cnt_ref, x_ref, wl_ref, wr_ref, b_ref, o_ref):
    cnt = cnt_ref[0][:, 0:1] + cnt_ref[1][:, 0:1]
    agg = (p_ref[0] + p_ref[1]) * (1.0 / jnp.maximum(cnt, 1.0))
    h = (jnp.dot(agg, wl_ref[...], preferred_element_type=jnp.float32)
         + jnp.dot(x_ref[...], wr_ref[...], preferred_element_type=jnp.float32)
         + b_ref[...])
    o_ref[...] = jnp.maximum(h, 0.0) if relu else h


def _tc_layer(p, cnt, x, wlT, wrT, bl, relu):
    return pl.pallas_call(
        functools.partial(_tc_layer_body, relu),
        grid=(TGRID,),
        in_specs=[
            pl.BlockSpec((NC, TB, D), lambda i: (0, i, 0)),
            pl.BlockSpec((NC, TB, CNT_W), lambda i: (0, i, 0)),
            pl.BlockSpec((TB, D), lambda i: (i, 0)),
            pl.BlockSpec((D, D), lambda i: (0, 0)),
            pl.BlockSpec((D, D), lambda i: (0, 0)),
            pl.BlockSpec((1, D), lambda i: (0, 0)),
        ],
        out_specs=pl.BlockSpec((TB, D), lambda i: (i, 0)),
        out_shape=jax.ShapeDtypeStruct((N, D), jnp.float32),
    )(p, cnt, x, wlT, wrT, bl)


def _tc_pool_body(p_ref, cnt_ref, x_ref, seg_ref, wl_ref, wr_ref, b_ref,
                  o_ref, acc, cac):
    i = pl.program_id(0)
    cnt = cnt_ref[0][:, 0:1] + cnt_ref[1][:, 0:1]
    agg = (p_ref[0] + p_ref[1]) * (1.0 / jnp.maximum(cnt, 1.0))
    h = (jnp.dot(agg, wl_ref[...], preferred_element_type=jnp.float32)
         + jnp.dot(x_ref[...], wr_ref[...], preferred_element_type=jnp.float32)
         + b_ref[...])
    oh = (seg_ref[...] == lax.broadcasted_iota(jnp.int32, (TB, G), 1)
          ).astype(jnp.float32)
    dn = (((0,), (0,)), ((), ()))
    part = lax.dot_general(oh, h, dn, preferred_element_type=jnp.float32)
    pcnt = lax.dot_general(oh, jnp.ones((TB, D), jnp.float32), dn,
                           preferred_element_type=jnp.float32)

    @pl.when(i == 0)
    def _():
        acc[...] = jnp.zeros((G, D), jnp.float32)
        cac[...] = jnp.zeros((G, D), jnp.float32)

    acc[...] += part
    cac[...] += pcnt

    @pl.when(i == TGRID - 1)
    def _():
        o_ref[...] = acc[...] / jnp.maximum(cac[...], 1.0)


def _tc_pool(p, cnt, x, seg, wlT, wrT, bl):
    return pl.pallas_call(
        _tc_pool_body,
        grid=(TGRID,),
        in_specs=[
            pl.BlockSpec((NC, TB, D), lambda i: (0, i, 0)),
            pl.BlockSpec((NC, TB, CNT_W), lambda i: (0, i, 0)),
            pl.BlockSpec((TB, D), lambda i: (i, 0)),
            pl.BlockSpec((TB, 1), lambda i: (i, 0)),
            pl.BlockSpec((D, D), lambda i: (0, 0)),
            pl.BlockSpec((D, D), lambda i: (0, 0)),
            pl.BlockSpec((1, D), lambda i: (0, 0)),
        ],
        out_specs=pl.BlockSpec((G, D), lambda i: (0, 0)),
        out_shape=jax.ShapeDtypeStruct((G, D), jnp.float32),
        scratch_shapes=[
            pltpu.VMEM((G, D), jnp.float32),
            pltpu.VMEM((G, D), jnp.float32),
        ],
    )(p, cnt, x, seg, wlT, wrT, bl)


def kernel(x, edge_index, batch, edge_attr,
           Wl0, bl0, Wr0, Wl1, bl1, Wr1, Wl2, bl2, Wr2):
    x = x.astype(jnp.float32)
    ei = edge_index.astype(jnp.int32)
    stripe = jnp.arange(PAD_E, dtype=jnp.int32)
    src = jnp.concatenate(
        [ei[0], stripe % jnp.int32(N)]).reshape(PCHUNK, 1, CH)
    dst = jnp.concatenate(
        [ei[1], jnp.int32(N) + stripe % jnp.int32(NPAD)]).reshape(PCHUNK, 1, CH)
    idx = jnp.concatenate(
        [src, dst], axis=1).reshape(PCHUNK // IB, IB, 2, CH)
    seg = batch.astype(jnp.int32).reshape(N, 1)

    cnt = _sc_cnt(idx)
    p = _sc_agg(x, idx)
    x1 = _tc_layer(p, cnt, x, Wl0.T, Wr0.T, bl0.reshape(1, D), relu=True)
    p = _sc_agg(x1, idx)
    x2 = _tc_layer(p, cnt, x1, Wl1.T, Wr1.T, bl1.reshape(1, D), relu=True)
    p = _sc_agg(x2, idx)
    return _tc_pool(p, cnt, x2, seg, Wl2.T, Wr2.T, bl2.reshape(1, D))


# speedup vs baseline: 2.5829x; 1.0700x over previous
import functools

import jax
import jax.numpy as jnp
from jax import lax
from jax.experimental import pallas as pl
from jax.experimental.pallas import tpu as pltpu
from jax.experimental.pallas import tpu_sc as plsc

N = 10000
E = 320000
D = 128
G = 64

NC = 2
NS = 16
NW = NC * NS
CH = 128
QT = 80
PCHUNK = NW * QT
IB = 16
QB = QT // IB
PAD_E = PCHUNK * CH - E
NPAD = 128
ZR = 200
NZB = N // ZR
OUT_SUB = 40
CNT_W = 16

_mesh = plsc.VectorSubcoreMesh(core_axis_name="c", subcore_axis_name="s")
_SC_PARAMS = pltpu.CompilerParams(use_tc_tiling_on_sc=False)


def _over_blocks(s, fn):
    def body(b, _):
        cid = s + b * NS

        @pl.when(cid < NZB)
        def _():
            fn(pl.multiple_of(cid * ZR, ZR))

        return 0

    lax.fori_loop(0, (NZB + NS - 1) // NS, body, 0)


def _sc_cnt_body(idx_hbm, cnt_hbm, cnt_sh, idx_v, ones_v, zcnt_v):
    c = lax.axis_index("c")
    s = lax.axis_index("s")
    wid = s * NC + c

    zf = jnp.zeros((16,), jnp.float32)
    one = jnp.ones((16,), jnp.float32)

    def fill(i, _):
        ones_v[i, :] = one

        @pl.when(i < OUT_SUB)
        def _():
            zcnt_v[i, :] = zf

        return 0

    lax.fori_loop(0, CH, fill, 0)

    def do_zero(r0):
        def sub(j, _):
            rr = pl.multiple_of(r0 + j * OUT_SUB, 8)
            pltpu.sync_copy(zcnt_v, cnt_sh.at[pl.ds(rr, OUT_SUB)])
            return 0

        lax.fori_loop(0, ZR // OUT_SUB, sub, 0)

    _over_blocks(s, do_zero)
    plsc.subcore_barrier()

    def chunk_block(u, _):
        bid = wid + u * NW
        pltpu.sync_copy(idx_hbm.at[bid], idx_v)
        for j in range(IB):
            pltpu.sync_copy(ones_v, cnt_sh.at[idx_v.at[j, 1]], add=True)
        return 0

    lax.fori_loop(0, QB, chunk_block, 0)

    plsc.subcore_barrier()

    def do_out(r0):
        def sub(j, _):
            rr = pl.multiple_of(r0 + j * OUT_SUB, 8)
            pltpu.sync_copy(cnt_sh.at[pl.ds(rr, OUT_SUB)],
                            cnt_hbm.at[c, pl.ds(rr, OUT_SUB)])
            return 0

        lax.fori_loop(0, ZR // OUT_SUB, sub, 0)

    _over_blocks(s, do_out)


_sc_cnt = pl.kernel(
    _sc_cnt_body,
    out_type=jax.ShapeDtypeStruct((NC, N, CNT_W), jnp.float32),
    mesh=_mesh,
    compiler_params=_SC_PARAMS,
    scratch_types=[
        pltpu.VMEM_SHARED((N + NPAD, CNT_W), jnp.float32),
        pltpu.VMEM((IB, 2, CH), jnp.int32),
        pltpu.VMEM((CH, CNT_W), jnp.float32),
        pltpu.VMEM((OUT_SUB, CNT_W), jnp.float32),
    ],
)

def _sc_body_dbuf(x_hbm, idx_hbm, out_hbm, agg_sh, idx_v, rows0, rows1,
                  sem0, sem1):
    c = lax.axis_index("c")
    s = lax.axis_index("s")
    wid = s * NC + c

    zf = jnp.zeros((16,), jnp.float32)

    def fill_zrow(i, _):
        for j in range(D // 16):
            rows0[i, pl.ds(j * 16, 16)] = zf
        return 0

    lax.fori_loop(0, CH, fill_zrow, 0)

    def do_zero(r0):
        pltpu.sync_copy(rows0.at[pl.ds(0, CH)], agg_sh.at[pl.ds(r0, CH)])
        pltpu.sync_copy(rows0.at[pl.ds(0, ZR - CH)],
                        agg_sh.at[pl.ds(r0 + CH, ZR - CH)])

    _over_blocks(s, do_zero)
    plsc.subcore_barrier()

    bufs = ((rows0, sem0), (rows1, sem1))

    def chunk_block(u, _):
        bid = wid + u * NW
        pltpu.sync_copy(idx_hbm.at[bid], idx_v)
        pltpu.async_copy(x_hbm.at[idx_v.at[0, 0]], rows0, sem0)
        for j in range(IB):
            rv, sm = bufs[j % 2]
            if j + 1 < IB:
                nrv, nsm = bufs[(j + 1) % 2]
                pltpu.async_copy(x_hbm.at[idx_v.at[j + 1, 0]], nrv, nsm)
            pltpu.make_async_copy(x_hbm.at[idx_v.at[j, 0]], rv, sm).wait()
            pltpu.sync_copy(rv, agg_sh.at[idx_v.at[j, 1]], add=True)
        return 0

    lax.fori_loop(0, QB, chunk_block, 0)

    plsc.subcore_barrier()

    def do_out(r0):
        def sub(j, _):
            rr = pl.multiple_of(r0 + j * OUT_SUB, 8)
            pltpu.sync_copy(agg_sh.at[pl.ds(rr, OUT_SUB)],
                            out_hbm.at[c, pl.ds(rr, OUT_SUB)])
            return 0

        lax.fori_loop(0, ZR // OUT_SUB, sub, 0)

    _over_blocks(s, do_out)


_sc_agg = pl.kernel(
    _sc_body_dbuf,
    out_type=jax.ShapeDtypeStruct((NC, N, D), jnp.float32),
    mesh=_mesh,
    compiler_params=_SC_PARAMS,
    scratch_types=[
        pltpu.VMEM_SHARED((N + NPAD, D), jnp.float32),
        pltpu.VMEM((IB, 2, CH), jnp.int32),
        pltpu.VMEM((CH, D), jnp.float32),
        pltpu.VMEM((CH, D), jnp.float32),
        pltpu.SemaphoreType.DMA,
        pltpu.SemaphoreType.DMA,
    ],
)


TB = 1000
TGRID = N // TB


def _tc_layer_body(relu, p_ref, cnt_ref, x_ref, wl_ref, wr_ref, b_ref, o_ref):
    cnt = cnt_ref[0][:, 0:1] + cnt_ref[1][:, 0:1]
    agg = (p_ref[0] + p_ref[1]) * (1.0 / jnp.maximum(cnt, 1.0))
    h = (jnp.dot(agg, wl_ref[...], preferred_element_type=jnp.float32)
         + jnp.dot(x_ref[...], wr_ref[...], preferred_element_type=jnp.float32)
         + b_ref[...])
    o_ref[...] = jnp.maximum(h, 0.0) if relu else h


def _tc_layer(p, cnt, x, wlT, wrT, bl, relu):
    return pl.pallas_call(
        functools.partial(_tc_layer_body, relu),
        grid=(TGRID,),
        in_specs=[
            pl.BlockSpec((NC, TB, D), lambda i: (0, i, 0)),
            pl.BlockSpec((NC, TB, CNT_W), lambda i: (0, i, 0)),
            pl.BlockSpec((TB, D), lambda i: (i, 0)),
            pl.BlockSpec((D, D), lambda i: (0, 0)),
            pl.BlockSpec((D, D), lambda i: (0, 0)),
            pl.BlockSpec((1, D), lambda i: (0, 0)),
        ],
        out_specs=pl.BlockSpec((TB, D), lambda i: (i, 0)),
        out_shape=jax.ShapeDtypeStruct((N, D), jnp.float32),
    )(p, cnt, x, wlT, wrT, bl)


def _tc_pool_body(p_ref, cnt_ref, x_ref, seg_ref, wl_ref, wr_ref, b_ref,
                  o_ref, acc, cac):
    i = pl.program_id(0)
    cnt = cnt_ref[0][:, 0:1] + cnt_ref[1][:, 0:1]
    agg = (p_ref[0] + p_ref[1]) * (1.0 / jnp.maximum(cnt, 1.0))
    h = (jnp.dot(agg, wl_ref[...], preferred_element_type=jnp.float32)
         + jnp.dot(x_ref[...], wr_ref[...], preferred_element_type=jnp.float32)
         + b_ref[...])
    oh = (seg_ref[...] == lax.broadcasted_iota(jnp.int32, (TB, G), 1)
          ).astype(jnp.float32)
    dn = (((0,), (0,)), ((), ()))
    part = lax.dot_general(oh, h, dn, preferred_element_type=jnp.float32)
    pcnt = lax.dot_general(oh, jnp.ones((TB, D), jnp.float32), dn,
                           preferred_element_type=jnp.float32)

    @pl.when(i == 0)
    def _():
        acc[...] = jnp.zeros((G, D), jnp.float32)
        cac[...] = jnp.zeros((G, D), jnp.float32)

    acc[...] += part
    cac[...] += pcnt

    @pl.when(i == TGRID - 1)
    def _():
        o_ref[...] = acc[...] / jnp.maximum(cac[...], 1.0)


def _tc_pool(p, cnt, x, seg, wlT, wrT, bl):
    return pl.pallas_call(
        _tc_pool_body,
        grid=(TGRID,),
        in_specs=[
            pl.BlockSpec((NC, TB, D), lambda i: (0, i, 0)),
            pl.BlockSpec((NC, TB, CNT_W), lambda i: (0, i, 0)),
            pl.BlockSpec((TB, D), lambda i: (i, 0)),
            pl.BlockSpec((TB, 1), lambda i: (i, 0)),
            pl.BlockSpec((D, D), lambda i: (0, 0)),
            pl.BlockSpec((D, D), lambda i: (0, 0)),
            pl.BlockSpec((1, D), lambda i: (0, 0)),
        ],
        out_specs=pl.BlockSpec((G, D), lambda i: (0, 0)),
        out_shape=jax.ShapeDtypeStruct((G, D), jnp.float32),
        scratch_shapes=[
            pltpu.VMEM((G, D), jnp.float32),
            pltpu.VMEM((G, D), jnp.float32),
        ],
    )(p, cnt, x, seg, wlT, wrT, bl)


def kernel(x, edge_index, batch, edge_attr,
           Wl0, bl0, Wr0, Wl1, bl1, Wr1, Wl2, bl2, Wr2):
    x = x.astype(jnp.float32)
    ei = edge_index.astype(jnp.int32)
    stripe = jnp.arange(PAD_E, dtype=jnp.int32)
    src = jnp.concatenate(
        [ei[0], stripe % jnp.int32(N)]).reshape(PCHUNK, 1, CH)
    dst = jnp.concatenate(
        [ei[1], jnp.int32(N) + stripe % jnp.int32(NPAD)]).reshape(PCHUNK, 1, CH)
    idx = jnp.concatenate(
        [src, dst], axis=1).reshape(PCHUNK // IB, IB, 2, CH)
    seg = batch.astype(jnp.int32).reshape(N, 1)

    cnt = _sc_cnt(idx)
    p = _sc_agg(x, idx)
    x1 = _tc_layer(p, cnt, x, Wl0.T, Wr0.T, bl0.reshape(1, D), relu=True)
    p = _sc_agg(x1, idx)
    x2 = _tc_layer(p, cnt, x1, Wl1.T, Wr1.T, bl1.reshape(1, D), relu=True)
    p = _sc_agg(x2, idx)
    return _tc_pool(p, cnt, x2, seg, Wl2.T, Wr2.T, bl2.reshape(1, D))
